# baseline (speedup 1.0000x reference)
import functools

import jax
import jax.numpy as jnp
import numpy as np
from jax.experimental import pallas as pl
from jax.experimental.pallas import tpu as pltpu

HIDDEN = 128
G = 128
N_LIG = 2048
N_PROT = 16384
PBLK = 4096
SUB = 4096
LHALF = N_LIG // 4
N_STEPS = N_PROT // PBLK


def _fused_kernel(lig_in, prot_in, Wl1, bl1, Wl2, bl2, Wp1, bp1, Wp2, bp2,
                  Wq, Wk, Wv, Wo, batch, out_ref,
                  q_s, emb_s, acc_s, m_s, l_s, w_s):
    i = pl.program_id(0)

    @pl.when(i == 0)
    def _prologue():
        h = jax.nn.silu(
            jnp.dot(lig_in[...], Wl1[...], preferred_element_type=jnp.float32)
            + bl1[...])
        emb = jnp.dot(h, Wl2[...], preferred_element_type=jnp.float32) + bl2[...]
        emb_s[...] = emb
        wpk = jnp.dot(Wp2[...], Wk[...], preferred_element_type=jnp.float32)
        wqpk = jax.lax.dot_general(
            Wq[...], wpk, (((1,), (1,)), ((), ())),
            preferred_element_type=jnp.float32) * (
                np.float32(np.log2(np.e) / np.sqrt(HIDDEN)))
        q_s[...] = jnp.dot(emb, wqpk, preferred_element_type=jnp.float32)
        m_s[...] = jnp.full_like(m_s, -jnp.inf)
        l_s[...] = jnp.zeros_like(l_s)
        acc_s[...] = jnp.zeros_like(acc_s)
        seg = jax.lax.broadcasted_iota(jnp.int32, (N_LIG, G), 1)
        onehot = (batch[...] == seg).astype(jnp.float32)
        counts = jnp.sum(onehot, axis=0, keepdims=True)
        w_s[...] = onehot / jnp.maximum(counts, 1.0)

    q = q_s[...]
    for j in range(PBLK // SUB):
        pin = prot_in[j * SUB:(j + 1) * SUB, :]
        ph = jax.nn.silu(
            jnp.dot(pin, Wp1[...], preferred_element_type=jnp.float32)
            + bp1[...])
        for h in range(N_LIG // LHALF):
            r = pl.ds(h * LHALF, LHALF)
            s = jax.lax.dot_general(q[h * LHALF:(h + 1) * LHALF, :], ph,
                                    (((1,), (1,)), ((), ())),
                                    preferred_element_type=jnp.float32)
            m_cur = jnp.max(s, axis=-1, keepdims=True)
            m_prev = m_s[r, :]
            m_new = jnp.maximum(m_prev, m_cur)
            p = jnp.exp2(s - m_new)
            alpha = jnp.exp2(m_prev - m_new)
            l_s[r, :] = l_s[r, :] * alpha + jnp.sum(p, axis=-1, keepdims=True)
            acc_s[r, :] = acc_s[r, :] * alpha + jnp.dot(
                p, ph, preferred_element_type=jnp.float32)
            m_s[r, :] = m_new

    @pl.when(i == N_STEPS - 1)
    def _epilogue():
        wvo = jnp.dot(Wv[...], Wo[...], preferred_element_type=jnp.float32)
        wpvo = jnp.dot(Wp2[...], wvo, preferred_element_type=jnp.float32)
        cvec = jnp.dot(bp2[...], wvo, preferred_element_type=jnp.float32)
        lig_out = emb_s[...] + cvec + jnp.dot(
            acc_s[...] / l_s[...], wpvo, preferred_element_type=jnp.float32)
        out_ref[...] = jax.lax.dot_general(
            w_s[...], lig_out, (((0,), (0,)), ((), ())),
            preferred_element_type=jnp.float32)


@jax.jit
def _run(lig_in, prot_in, Wl1, bl1, Wl2, bl2, Wp1, bp1, Wp2, bp2,
         Wq, Wk, Wv, Wo, batch):
    full = lambda shape: pl.BlockSpec(shape, lambda i: (0,) * len(shape))
    return pl.pallas_call(
        _fused_kernel,
        grid=(N_STEPS,),
        in_specs=[
            full((N_LIG, 13)),
            pl.BlockSpec((PBLK, 30), lambda i: (i, 0)),
            full((13, HIDDEN)), full((1, HIDDEN)),
            full((HIDDEN, HIDDEN)), full((1, HIDDEN)),
            full((30, HIDDEN)), full((1, HIDDEN)),
            full((HIDDEN, HIDDEN)), full((1, HIDDEN)),
            full((HIDDEN, HIDDEN)), full((HIDDEN, HIDDEN)),
            full((HIDDEN, HIDDEN)), full((HIDDEN, HIDDEN)),
            full((N_LIG, 1)),
        ],
        out_specs=full((G, HIDDEN)),
        out_shape=jax.ShapeDtypeStruct((G, HIDDEN), jnp.float32),
        scratch_shapes=[
            pltpu.VMEM((N_LIG, HIDDEN), jnp.float32),
            pltpu.VMEM((N_LIG, HIDDEN), jnp.float32),
            pltpu.VMEM((N_LIG, HIDDEN), jnp.float32),
            pltpu.VMEM((N_LIG, 1), jnp.float32),
            pltpu.VMEM((N_LIG, 1), jnp.float32),
            pltpu.VMEM((N_LIG, G), jnp.float32),
        ],
    )(lig_in, prot_in, Wl1, bl1, Wl2, bl2, Wp1, bp1, Wp2, bp2,
      Wq, Wk, Wv, Wo, batch)


def kernel(ligand_atom_feature, ligand_pos, protein_atom_feature_full,
           protein_pos, Wl1, bl1, Wl2, bl2, Wp1, bp1, Wp2, bp2,
           Wq, Wk, Wv, Wo, ligand_atom_feature_batch,
           protein_atom_feature_full_batch):
    lig_in = jnp.concatenate(
        [ligand_atom_feature.astype(jnp.float32), ligand_pos], axis=-1)
    prot_in = jnp.concatenate(
        [protein_atom_feature_full.astype(jnp.float32), protein_pos], axis=-1)
    batch = ligand_atom_feature_batch.astype(jnp.int32).reshape(N_LIG, 1)
    return _run(lig_in, prot_in,
                Wl1, bl1.reshape(1, HIDDEN), Wl2, bl2.reshape(1, HIDDEN),
                Wp1, bp1.reshape(1, HIDDEN), Wp2, bp2.reshape(1, HIDDEN),
                Wq, Wk, Wv, Wo, batch)


# speedup vs baseline: 1.0128x; 1.0128x over previous
import jax
import jax.numpy as jnp
import numpy as np
from jax.experimental import pallas as pl
from jax.experimental.pallas import tpu as pltpu

HIDDEN = 128
G = 128
N_LIG = 2048
N_PROT = 16384
PBLK = 4096
SUB = 4096
LHALF = N_LIG // 2
N_STEPS = N_PROT // PBLK


def _fused_kernel(lig_in, prot_in, Wl1, bl1, Wl2, bl2, Wp1, bp1, Wp2, bp2,
                  Wq, Wk, Wv, Wo, batch, out_ref,
                  q_s, emb_s, acc_s, m_s, l_s, w_s):
    i = pl.program_id(0)

    @pl.when(i == 0)
    def _prologue():
        h = jax.nn.silu(
            jnp.dot(lig_in[...], Wl1[...], preferred_element_type=jnp.float32)
            + bl1[...])
        emb = jnp.dot(h, Wl2[...], preferred_element_type=jnp.float32) + bl2[...]
        emb_s[...] = emb
        wpk = jnp.dot(Wp2[...], Wk[...], preferred_element_type=jnp.float32)
        wqpk = jax.lax.dot_general(
            Wq[...], wpk, (((1,), (1,)), ((), ())),
            preferred_element_type=jnp.float32) * (
                np.float32(np.log2(np.e) / np.sqrt(HIDDEN)))
        q_s[...] = jnp.dot(emb, wqpk, preferred_element_type=jnp.float32)
        m_s[...] = jnp.full_like(m_s, -jnp.inf)
        l_s[...] = jnp.zeros_like(l_s)
        acc_s[...] = jnp.zeros_like(acc_s)
        seg = jax.lax.broadcasted_iota(jnp.int32, (N_LIG, G), 1)
        onehot = (batch[...] == seg).astype(jnp.float32)
        counts = jnp.sum(onehot, axis=0, keepdims=True)
        w_s[...] = onehot / jnp.maximum(counts, 1.0)

    q = q_s[...]
    for j in range(PBLK // SUB):
        pin = prot_in[j * SUB:(j + 1) * SUB, :]
        ph = jax.nn.silu(
            jnp.dot(pin, Wp1[...], preferred_element_type=jnp.float32)
            + bp1[...])
        for h in range(N_LIG // LHALF):
            r = pl.ds(h * LHALF, LHALF)
            s = jax.lax.dot_general(q[h * LHALF:(h + 1) * LHALF, :], ph,
                                    (((1,), (1,)), ((), ())),
                                    preferred_element_type=jnp.float32)
            m_cur = jnp.max(s, axis=-1, keepdims=True)
            m_prev = m_s[r, :]
            m_new = jnp.maximum(m_prev, m_cur)
            p = jnp.exp2(s - m_new)
            alpha = jnp.exp2(m_prev - m_new)
            l_s[r, :] = l_s[r, :] * alpha + jnp.sum(p, axis=-1, keepdims=True)
            acc_s[r, :] = acc_s[r, :] * alpha + jnp.dot(
                p, ph, preferred_element_type=jnp.float32)
            m_s[r, :] = m_new

    @pl.when(i == N_STEPS - 1)
    def _epilogue():
        wvo = jnp.dot(Wv[...], Wo[...], preferred_element_type=jnp.float32)
        wpvo = jnp.dot(Wp2[...], wvo, preferred_element_type=jnp.float32)
        cvec = jnp.dot(bp2[...], wvo, preferred_element_type=jnp.float32)
        lig_out = emb_s[...] + cvec + jnp.dot(
            acc_s[...] / l_s[...], wpvo, preferred_element_type=jnp.float32)
        out_ref[...] = jax.lax.dot_general(
            w_s[...], lig_out, (((0,), (0,)), ((), ())),
            preferred_element_type=jnp.float32)


@jax.jit
def _run(lig_in, prot_in, Wl1, bl1, Wl2, bl2, Wp1, bp1, Wp2, bp2,
         Wq, Wk, Wv, Wo, batch):
    full = lambda shape: pl.BlockSpec(shape, lambda i: (0,) * len(shape))
    return pl.pallas_call(
        _fused_kernel,
        grid=(N_STEPS,),
        in_specs=[
            full((N_LIG, 13)),
            pl.BlockSpec((PBLK, 30), lambda i: (i, 0)),
            full((13, HIDDEN)), full((1, HIDDEN)),
            full((HIDDEN, HIDDEN)), full((1, HIDDEN)),
            full((30, HIDDEN)), full((1, HIDDEN)),
            full((HIDDEN, HIDDEN)), full((1, HIDDEN)),
            full((HIDDEN, HIDDEN)), full((HIDDEN, HIDDEN)),
            full((HIDDEN, HIDDEN)), full((HIDDEN, HIDDEN)),
            full((N_LIG, 1)),
        ],
        out_specs=full((G, HIDDEN)),
        out_shape=jax.ShapeDtypeStruct((G, HIDDEN), jnp.float32),
        scratch_shapes=[
            pltpu.VMEM((N_LIG, HIDDEN), jnp.float32),
            pltpu.VMEM((N_LIG, HIDDEN), jnp.float32),
            pltpu.VMEM((N_LIG, HIDDEN), jnp.float32),
            pltpu.VMEM((N_LIG, 1), jnp.float32),
            pltpu.VMEM((N_LIG, 1), jnp.float32),
            pltpu.VMEM((N_LIG, G), jnp.float32),
        ],
    )(lig_in, prot_in, Wl1, bl1, Wl2, bl2, Wp1, bp1, Wp2, bp2,
      Wq, Wk, Wv, Wo, batch)


def kernel(ligand_atom_feature, ligand_pos, protein_atom_feature_full,
           protein_pos, Wl1, bl1, Wl2, bl2, Wp1, bp1, Wp2, bp2,
           Wq, Wk, Wv, Wo, ligand_atom_feature_batch,
           protein_atom_feature_full_batch):
    lig_in = jnp.concatenate(
        [ligand_atom_feature.astype(jnp.float32), ligand_pos], axis=-1)
    prot_in = jnp.concatenate(
        [protein_atom_feature_full.astype(jnp.float32), protein_pos], axis=-1)
    batch = ligand_atom_feature_batch.astype(jnp.int32).reshape(N_LIG, 1)
    return _run(lig_in, prot_in,
                Wl1, bl1.reshape(1, HIDDEN), Wl2, bl2.reshape(1, HIDDEN),
                Wp1, bp1.reshape(1, HIDDEN), Wp2, bp2.reshape(1, HIDDEN),
                Wq, Wk, Wv, Wo, batch)
